# baseline (speedup 1.0000x reference)
import jax
import jax.numpy as jnp
from jax import lax
from jax.experimental import pallas as pl
from jax.experimental.pallas import tpu as pltpu
from jax.experimental.pallas import tpu_sc as plsc

NUM_FACTORS = 32
BATCH = 16384
NC = 2
NS = 16
L = 16
NW = NC * NS
B_PER_W = BATCH // NW
CHUNK = 128
N_CHUNKS = B_PER_W // CHUNK
G_PER_CHUNK = CHUNK // L


def _mf_body(uid_hbm, aid_hbm, uidx4_hbm, aidx4_hbm, uoff_hbm, aoff_hbm,
             uemb_hbm, aemb_hbm, ubias_hbm, abias_hbm,
             out_hbm, uidx_v, aidx_v, uidx4_v, aidx4_v, uoff_v, aoff_v,
             ubuf, abuf, ub1, ab1, out_v,
             sem_u0, sem_u1, sem_a0, sem_a1, sem_b):
    wid = lax.axis_index("s") * NC + lax.axis_index("c")
    base = wid * B_PER_W

    pltpu.sync_copy(uid_hbm.at[wid], uidx_v)
    pltpu.sync_copy(aid_hbm.at[wid], aidx_v)
    pltpu.sync_copy(uidx4_hbm.at[wid], uidx4_v)
    pltpu.sync_copy(aidx4_hbm.at[wid], aidx4_v)
    pltpu.sync_copy(uoff_hbm.at[wid], uoff_v)
    pltpu.sync_copy(aoff_hbm.at[wid], aoff_v)

    sems_u = (sem_u0, sem_u1)
    sems_a = (sem_a0, sem_a1)

    def fire(c):
        idx = pl.ds(c * CHUNK, CHUNK)
        cu = pltpu.async_copy(uemb_hbm.at[uidx4_v.at[idx]],
                              ubuf.at[c % 2], sems_u[c % 2])
        ca = pltpu.async_copy(aemb_hbm.at[aidx4_v.at[idx]],
                              abuf.at[c % 2], sems_a[c % 2])
        return cu, ca

    bias_copies = []
    for c in range(N_CHUNKS):
        idx = pl.ds(c * CHUNK, CHUNK)
        bias_copies.append(pltpu.async_copy(ubias_hbm.at[uidx_v.at[idx]],
                                            ub1.at[idx], sem_b))
        bias_copies.append(pltpu.async_copy(abias_hbm.at[aidx_v.at[idx]],
                                            ab1.at[idx], sem_b))
    pending = fire(0)
    for c in bias_copies:
        c.wait()

    iota16 = lax.iota(jnp.int32, L)

    for c in range(N_CHUNKS):
        pending[0].wait()
        pending[1].wait()
        if c + 1 < N_CHUNKS:
            pending = fire(c + 1)
        ub = ubuf.at[c % 2]
        ab = abuf.at[c % 2]
        for k in range(G_PER_CHUNK):
            e0 = c * CHUNK + k * L
            rows = k * L + iota16
            ucols = uoff_v[pl.ds(e0, L)]
            acols = aoff_v[pl.ds(e0, L)]
            acc = ub1[pl.ds(e0, L)] + ab1[pl.ds(e0, L)]
            for j in range(NUM_FACTORS):
                uu = plsc.load_gather(ub, [rows, ucols + j])
                aa = plsc.load_gather(ab, [rows, acols + j])
                acc = acc + uu * aa
            out_v[pl.ds(e0, L)] = acc

    pltpu.sync_copy(out_v, out_hbm.at[pl.ds(base, B_PER_W)])


@jax.jit
def _mf_kernel(uids, aids, uidx4, aidx4, uoff, aoff, ue2, ae2, ub1, ab1):
    mesh = plsc.VectorSubcoreMesh(core_axis_name="c", subcore_axis_name="s",
                                  num_cores=NC, num_subcores=NS)
    return pl.kernel(
        _mf_body,
        out_type=jax.ShapeDtypeStruct((BATCH,), jnp.float32),
        mesh=mesh,
        compiler_params=pltpu.CompilerParams(needs_layout_passes=False),
        scratch_types=[
            pltpu.VMEM((B_PER_W,), jnp.int32),
            pltpu.VMEM((B_PER_W,), jnp.int32),
            pltpu.VMEM((B_PER_W,), jnp.int32),
            pltpu.VMEM((B_PER_W,), jnp.int32),
            pltpu.VMEM((B_PER_W,), jnp.int32),
            pltpu.VMEM((B_PER_W,), jnp.int32),
            pltpu.VMEM((2, CHUNK, 4 * NUM_FACTORS), jnp.float32),
            pltpu.VMEM((2, CHUNK, 4 * NUM_FACTORS), jnp.float32),
            pltpu.VMEM((B_PER_W,), jnp.float32),
            pltpu.VMEM((B_PER_W,), jnp.float32),
            pltpu.VMEM((B_PER_W,), jnp.float32),
            pltpu.SemaphoreType.DMA,
            pltpu.SemaphoreType.DMA,
            pltpu.SemaphoreType.DMA,
            pltpu.SemaphoreType.DMA,
            pltpu.SemaphoreType.DMA,
        ],
    )(uids, aids, uidx4, aidx4, uoff, aoff, ue2, ae2, ub1, ab1)


def kernel(userIds, animeIds, user_embeddings, anime_embeddings,
           user_biases, anime_biases):
    uids = userIds.astype(jnp.int32)
    aids = animeIds.astype(jnp.int32)
    uidx4 = (uids >> 2).reshape(NW, B_PER_W)
    aidx4 = (aids >> 2).reshape(NW, B_PER_W)
    uoff = ((uids & 3) * NUM_FACTORS).reshape(NW, B_PER_W)
    aoff = ((aids & 3) * NUM_FACTORS).reshape(NW, B_PER_W)
    ue2 = user_embeddings.reshape(-1, 4 * NUM_FACTORS)
    ae2 = anime_embeddings.reshape(-1, 4 * NUM_FACTORS)
    ub1 = user_biases.reshape(-1)
    ab1 = anime_biases.reshape(-1)
    return _mf_kernel(uids.reshape(NW, B_PER_W), aids.reshape(NW, B_PER_W),
                      uidx4, aidx4, uoff, aoff, ue2, ae2, ub1, ab1)
